# baseline (speedup 1.0000x reference)
import jax
import jax.numpy as jnp
from jax.experimental import pallas as pl
from jax.experimental.pallas import tpu as pltpu

_EPS = 1e-05
_NUM_NODES = 3
_NUM_LEAVES = 4

_A = ((1.0, 1.0, -1.0, -1.0),
      (1.0, -1.0, 0.0, 0.0),
      (0.0, 0.0, 1.0, -1.0))
_B = (1.5, 0.5, 0.5, -0.5)


def _predict_rows(x, oh, thr, leaf, a, b):
    xt = jax.lax.dot(x, oh, preferred_element_type=jnp.float32)
    c = ((xt + _EPS) < thr).astype(jnp.float32)
    t = jax.lax.dot(c, a, preferred_element_type=jnp.float32)
    w = (t > b).astype(jnp.float32)
    return jax.lax.dot(w, leaf, preferred_element_type=jnp.float32)


def _round_up(v, m):
    return ((v + m - 1) // m) * m


def _make_pipe_kernel(n, nb, cb):
    def _kernel(x_hbm, oh_ref, thr_ref, leaf_ref, a_ref, b_ref, out_hbm,
                x_vm, o_vm, in_sem, out_sem):
        j = pl.program_id(0)

        def start_in(chunk, slot):
            pltpu.make_async_copy(x_hbm.at[pl.ds(chunk * cb, cb)],
                                  x_vm.at[slot], in_sem.at[slot]).start()

        def wait_in(slot):
            pltpu.make_async_copy(x_hbm.at[pl.ds(0, cb)],
                                  x_vm.at[slot], in_sem.at[slot]).wait()

        def start_out(chunk, slot):
            pltpu.make_async_copy(o_vm.at[slot],
                                  out_hbm.at[pl.ds(chunk * cb, cb)],
                                  out_sem.at[slot]).start()

        def wait_out(slot):
            pltpu.make_async_copy(o_vm.at[slot],
                                  out_hbm.at[pl.ds(0, cb)],
                                  out_sem.at[slot]).wait()

        @pl.when(j == 0)
        def _():
            for k in range(min(nb, n)):
                start_in(k, k)

        if n > 1:
            @pl.when((j > 0) & (j + nb - 1 < n))
            def _():
                start_in(j + nb - 1, (j + nb - 1) % nb)

        slot = jax.lax.rem(j, nb)
        wait_in(slot)
        if n > nb:
            @pl.when(j >= nb)
            def _():
                wait_out(slot)

        a = a_ref[...]
        b = b_ref[...]
        for i in range(cb):
            o_vm[slot, i] = _predict_rows(x_vm[slot, i], oh_ref[i],
                                          thr_ref[i], leaf_ref[i], a, b)
        start_out(j, slot)

        @pl.when(j == n - 1)
        def _():
            for s in range(min(nb, n)):
                wait_out(s)

    return _kernel


def _predict_pipelined(xf, oh, thr, leaf, a_mat, b_row, nb, cb):
    bs, L, D = xf.shape
    Y = leaf.shape[-1]
    n = bs // cb
    return pl.pallas_call(
        _make_pipe_kernel(n, nb, cb),
        out_shape=jax.ShapeDtypeStruct((bs, L, Y), jnp.float32),
        grid=(n,),
        in_specs=[
            pl.BlockSpec(memory_space=pl.ANY),
            pl.BlockSpec((cb, D, _NUM_NODES), lambda j: (j, 0, 0)),
            pl.BlockSpec((cb, 1, _NUM_NODES), lambda j: (j, 0, 0)),
            pl.BlockSpec((cb, _NUM_LEAVES, Y), lambda j: (j, 0, 0)),
            pl.BlockSpec((_NUM_NODES, _NUM_LEAVES), lambda j: (0, 0)),
            pl.BlockSpec((1, _NUM_LEAVES), lambda j: (0, 0)),
        ],
        out_specs=pl.BlockSpec(memory_space=pl.ANY),
        scratch_shapes=[
            pltpu.VMEM((nb, cb, L, D), jnp.float32),
            pltpu.VMEM((nb, cb, L, Y), jnp.float32),
            pltpu.SemaphoreType.DMA((nb,)),
            pltpu.SemaphoreType.DMA((nb,)),
        ],
        compiler_params=pltpu.CompilerParams(
            dimension_semantics=("arbitrary",)),
    )(xf, oh, thr, leaf, a_mat, b_row)


def _tree_kernel(x_ref, oh_ref, thr_ref, leaf_ref, a_ref, b_ref, out_ref):
    a = a_ref[...]
    b = b_ref[...]
    for i in range(x_ref.shape[0]):
        out_ref[i] = _predict_rows(x_ref[i], oh_ref[i], thr_ref[i],
                                   leaf_ref[i], a, b)


def _predict_blocked(xf, oh, thr, leaf, a_mat, b_row):
    bs, L, D = xf.shape
    Y = leaf.shape[-1]

    l_blk = min(_round_up(L, 8), 1024)
    L_pad = _round_up(L, l_blk)
    n_l = L_pad // l_blk
    if L_pad != L:
        xf = jnp.pad(xf, ((0, 0), (0, L_pad - L), (0, 0)))

    b_blk = 1
    for cand in (16, 8, 4, 2):
        if bs % cand == 0:
            b_blk = cand
            break
    n_b = bs // b_blk

    out = pl.pallas_call(
        _tree_kernel,
        out_shape=jax.ShapeDtypeStruct((bs, L_pad, Y), jnp.float32),
        grid=(n_b, n_l),
        in_specs=[
            pl.BlockSpec((b_blk, l_blk, D), lambda b, l: (b, l, 0)),
            pl.BlockSpec((b_blk, D, _NUM_NODES), lambda b, l: (b, 0, 0)),
            pl.BlockSpec((b_blk, 1, _NUM_NODES), lambda b, l: (b, 0, 0)),
            pl.BlockSpec((b_blk, _NUM_LEAVES, Y), lambda b, l: (b, 0, 0)),
            pl.BlockSpec((_NUM_NODES, _NUM_LEAVES), lambda b, l: (0, 0)),
            pl.BlockSpec((1, _NUM_LEAVES), lambda b, l: (0, 0)),
        ],
        out_specs=pl.BlockSpec((b_blk, l_blk, Y), lambda b, l: (b, l, 0)),
        compiler_params=pltpu.CompilerParams(
            dimension_semantics=("parallel", "parallel")),
    )(xf, oh, thr, leaf, a_mat, b_row)
    return out[:, :L]


def kernel(x, onehot, thresholds, leaf_labels):
    bs, L, D = x.shape
    Y = leaf_labels.shape[-1]

    xf = x.astype(jnp.float32)
    oh = onehot.astype(jnp.float32)
    thr = thresholds.astype(jnp.float32)[:, None, :]
    leaf = leaf_labels.astype(jnp.float32)

    a_mat = jnp.array(_A, jnp.float32)
    b_row = jnp.array(_B, jnp.float32)[None, :]

    cb = 16
    if bs % cb == 0 and L % 8 == 0 and bs // cb >= 1:
        return _predict_pipelined(xf, oh, thr, leaf, a_mat, b_row, 2, cb)
    return _predict_blocked(xf, oh, thr, leaf, a_mat, b_row)


# speedup vs baseline: 1.0873x; 1.0873x over previous
import jax
import jax.numpy as jnp
from jax.experimental import pallas as pl
from jax.experimental.pallas import tpu as pltpu

_EPS = 1e-05
_NUM_NODES = 3
_NUM_LEAVES = 4

_A = ((1.0, 1.0, -1.0, -1.0),
      (1.0, -1.0, 0.0, 0.0),
      (0.0, 0.0, 1.0, -1.0))
_B = (1.5, 0.5, 0.5, -0.5)


def _predict_rows(x, oh, thr, leaf, a, b):
    xt = jax.lax.dot(x, oh, preferred_element_type=jnp.float32)
    c = ((xt + _EPS) < thr).astype(jnp.float32)
    t = jax.lax.dot(c, a, preferred_element_type=jnp.float32)
    w = (t > b).astype(jnp.float32)
    return jax.lax.dot(w, leaf, preferred_element_type=jnp.float32)


def _round_up(v, m):
    return ((v + m - 1) // m) * m


def _make_pipe_kernel(n, nb, cb):
    def _kernel(x_hbm, oh_ref, thr_ref, leaf_ref, a_ref, b_ref, out_hbm,
                x_vm, o_vm, in_sem, out_sem):
        j = pl.program_id(0)

        def start_in(chunk, slot):
            pltpu.make_async_copy(x_hbm.at[pl.ds(chunk * cb, cb)],
                                  x_vm.at[slot], in_sem.at[slot]).start()

        def wait_in(slot):
            pltpu.make_async_copy(x_hbm.at[pl.ds(0, cb)],
                                  x_vm.at[slot], in_sem.at[slot]).wait()

        def start_out(chunk, slot):
            pltpu.make_async_copy(o_vm.at[slot],
                                  out_hbm.at[pl.ds(chunk * cb, cb)],
                                  out_sem.at[slot]).start()

        def wait_out(slot):
            pltpu.make_async_copy(o_vm.at[slot],
                                  out_hbm.at[pl.ds(0, cb)],
                                  out_sem.at[slot]).wait()

        @pl.when(j == 0)
        def _():
            for k in range(min(nb, n)):
                start_in(k, k)

        if n > 1:
            @pl.when((j > 0) & (j + nb - 1 < n))
            def _():
                start_in(j + nb - 1, (j + nb - 1) % nb)

        slot = jax.lax.rem(j, nb)
        wait_in(slot)
        if n > nb:
            @pl.when(j >= nb)
            def _():
                wait_out(slot)

        a = a_ref[...]
        b = b_ref[...]
        for i in range(cb):
            o_vm[slot, i] = _predict_rows(x_vm[slot, i], oh_ref[i],
                                          thr_ref[i], leaf_ref[i], a, b)
        start_out(j, slot)

        @pl.when(j == n - 1)
        def _():
            for s in range(min(nb, n)):
                wait_out(s)

    return _kernel


def _predict_pipelined(xf, oh, thr, leaf, a_mat, b_row, nb, cb):
    bs, L, D = xf.shape
    Y = leaf.shape[-1]
    n = bs // cb
    return pl.pallas_call(
        _make_pipe_kernel(n, nb, cb),
        out_shape=jax.ShapeDtypeStruct((bs, L, Y), jnp.float32),
        grid=(n,),
        in_specs=[
            pl.BlockSpec(memory_space=pl.ANY),
            pl.BlockSpec((cb, D, _NUM_NODES), lambda j: (j, 0, 0)),
            pl.BlockSpec((cb, 1, _NUM_NODES), lambda j: (j, 0, 0)),
            pl.BlockSpec((cb, _NUM_LEAVES, Y), lambda j: (j, 0, 0)),
            pl.BlockSpec((_NUM_NODES, _NUM_LEAVES), lambda j: (0, 0)),
            pl.BlockSpec((1, _NUM_LEAVES), lambda j: (0, 0)),
        ],
        out_specs=pl.BlockSpec(memory_space=pl.ANY),
        scratch_shapes=[
            pltpu.VMEM((nb, cb, L, D), jnp.float32),
            pltpu.VMEM((nb, cb, L, Y), jnp.float32),
            pltpu.SemaphoreType.DMA((nb,)),
            pltpu.SemaphoreType.DMA((nb,)),
        ],
        compiler_params=pltpu.CompilerParams(
            dimension_semantics=("arbitrary",)),
    )(xf, oh, thr, leaf, a_mat, b_row)


def _tree_kernel(x_ref, oh_ref, thr_ref, leaf_ref, a_ref, b_ref, out_ref):
    a = a_ref[...]
    b = b_ref[...]
    for i in range(x_ref.shape[0]):
        out_ref[i] = _predict_rows(x_ref[i], oh_ref[i], thr_ref[i],
                                   leaf_ref[i], a, b)


def _predict_blocked(xf, oh, thr, leaf, a_mat, b_row):
    bs, L, D = xf.shape
    Y = leaf.shape[-1]

    l_blk = min(_round_up(L, 8), 1024)
    L_pad = _round_up(L, l_blk)
    n_l = L_pad // l_blk
    if L_pad != L:
        xf = jnp.pad(xf, ((0, 0), (0, L_pad - L), (0, 0)))

    b_blk = 1
    for cand in (16, 8, 4, 2):
        if bs % cand == 0:
            b_blk = cand
            break
    n_b = bs // b_blk

    out = pl.pallas_call(
        _tree_kernel,
        out_shape=jax.ShapeDtypeStruct((bs, L_pad, Y), jnp.float32),
        grid=(n_b, n_l),
        in_specs=[
            pl.BlockSpec((b_blk, l_blk, D), lambda b, l: (b, l, 0)),
            pl.BlockSpec((b_blk, D, _NUM_NODES), lambda b, l: (b, 0, 0)),
            pl.BlockSpec((b_blk, 1, _NUM_NODES), lambda b, l: (b, 0, 0)),
            pl.BlockSpec((b_blk, _NUM_LEAVES, Y), lambda b, l: (b, 0, 0)),
            pl.BlockSpec((_NUM_NODES, _NUM_LEAVES), lambda b, l: (0, 0)),
            pl.BlockSpec((1, _NUM_LEAVES), lambda b, l: (0, 0)),
        ],
        out_specs=pl.BlockSpec((b_blk, l_blk, Y), lambda b, l: (b, l, 0)),
        compiler_params=pltpu.CompilerParams(
            dimension_semantics=("parallel", "parallel")),
    )(xf, oh, thr, leaf, a_mat, b_row)
    return out[:, :L]


def kernel(x, onehot, thresholds, leaf_labels):
    bs, L, D = x.shape
    Y = leaf_labels.shape[-1]

    xf = x.astype(jnp.float32)
    oh = onehot.astype(jnp.float32)
    thr = thresholds.astype(jnp.float32)[:, None, :]
    leaf = leaf_labels.astype(jnp.float32)

    a_mat = jnp.array(_A, jnp.float32)
    b_row = jnp.array(_B, jnp.float32)[None, :]

    cb = 8
    if bs % cb == 0 and L % 8 == 0 and bs // cb >= 1:
        return _predict_pipelined(xf, oh, thr, leaf, a_mat, b_row, 4, cb)
    return _predict_blocked(xf, oh, thr, leaf, a_mat, b_row)
